# baseline (speedup 1.0000x reference)
import functools

import jax
import jax.numpy as jnp
from jax import lax
from jax.experimental import pallas as pl
from jax.experimental.pallas import tpu as pltpu
from jax.experimental.pallas import tpu_sc as plsc

LEVELS = 16
F_PER = 2
BASE_RES = 16
SCALE = 1.5
TABLE = 524288
MASK = TABLE - 1
RES = [int(BASE_RES * SCALE**l) for l in range(LEVELS)]

NC = 2
NS = 16
NW = NC * NS
LANES = 16

C = 256
G = C // LANES
HALVES = C // 128
ROWS = 4 * LEVELS


def _encoder_body(x_hbm, y_hbm, emb_hbm, out_hbm,
                  xv, yv, uxv, uyv, idx3, rowsf, out_buf, sem,
                  *, n_points):
    pw = n_points // NW
    nchunk = pw // C
    wid = lax.axis_index("s") * jnp.int32(NC) + lax.axis_index("c")
    base0 = wid * jnp.int32(pw)
    iota = lax.iota(jnp.int32, LANES)

    def chunk_body(ci, carry):
        base = base0 + ci * jnp.int32(C)
        pltpu.sync_copy(x_hbm.at[pl.ds(base, C)], xv)
        pltpu.sync_copy(y_hbm.at[pl.ds(base, C)], yv)

        def ubody(g, c_):
            sl = pl.ds(g * jnp.int32(LANES), LANES)
            uxv[sl] = jnp.clip((xv[sl] + 1.0) * 0.5, 0.0, 1.0)
            uyv[sl] = jnp.clip((yv[sl] + 1.0) * 0.5, 0.0, 1.0)
            return c_
        lax.fori_loop(jnp.int32(0), jnp.int32(G), ubody, jnp.int32(0))

        handles = []
        for l in range(LEVELS):
            resm1 = RES[l] - 1

            def hbody(g, c_, l=l, resm1=resm1):
                sl = pl.ds(g * jnp.int32(LANES), LANES)
                px = uxv[sl] * float(resm1)
                py = uyv[sl] * float(resm1)
                x0 = px.astype(jnp.int32)
                y0 = py.astype(jnp.int32)
                x1 = jnp.minimum(x0 + jnp.int32(1), jnp.int32(resm1))
                y1 = jnp.minimum(y0 + jnp.int32(1), jnp.int32(resm1))
                hx0 = x0 * jnp.int32(73856093)
                hx1 = x1 * jnp.int32(73856093)
                hy0 = y0 * jnp.int32(19349663)
                hy1 = y1 * jnp.int32(19349663)
                off = jnp.int32(l * TABLE)
                msk = jnp.int32(MASK)
                two = jnp.int32(2)
                one = jnp.int32(1)
                h00 = (((hx0 ^ hy0) & msk) + off) * two
                h10 = (((hx1 ^ hy0) & msk) + off) * two
                h01 = (((hx0 ^ hy1) & msk) + off) * two
                h11 = (((hx1 ^ hy1) & msk) + off) * two
                p0 = g * jnp.int32(LANES)
                p1 = p0 + jnp.int32(C)
                idx3[4 * l + 0, pl.ds(p0, LANES)] = h00
                idx3[4 * l + 0, pl.ds(p1, LANES)] = h00 + one
                idx3[4 * l + 1, pl.ds(p0, LANES)] = h10
                idx3[4 * l + 1, pl.ds(p1, LANES)] = h10 + one
                idx3[4 * l + 2, pl.ds(p0, LANES)] = h01
                idx3[4 * l + 2, pl.ds(p1, LANES)] = h01 + one
                idx3[4 * l + 3, pl.ds(p0, LANES)] = h11
                idx3[4 * l + 3, pl.ds(p1, LANES)] = h11 + one
                return c_
            lax.fori_loop(jnp.int32(0), jnp.int32(G), hbody, jnp.int32(0))

            for c in range(4):
                r = 4 * l + c
                h = pltpu.make_async_copy(emb_hbm.at[idx3.at[jnp.int32(r)]],
                                          rowsf.at[jnp.int32(r)], sem)
                h.start()
                handles.append(h)

        for h in handles:
            h.wait()

        for l in range(LEVELS):
            resm1 = RES[l] - 1

            def cbody(g, c_, l=l, resm1=resm1):
                sl = pl.ds(g * jnp.int32(LANES), LANES)
                px = uxv[sl] * float(resm1)
                py = uyv[sl] * float(resm1)
                wx = px - px.astype(jnp.int32).astype(jnp.float32)
                wy = py - py.astype(jnp.int32).astype(jnp.float32)
                omwx = 1.0 - wx
                omwy = 1.0 - wy
                pv = iota + (g * jnp.int32(LANES))
                for f in range(F_PER):
                    fs = pl.ds(g * jnp.int32(LANES) + jnp.int32(f * C), LANES)
                    e00 = rowsf[4 * l + 0, fs]
                    e10 = rowsf[4 * l + 1, fs]
                    e01 = rowsf[4 * l + 2, fs]
                    e11 = rowsf[4 * l + 3, fs]
                    ex0 = e00 * omwx + e10 * wx
                    ex1 = e01 * omwx + e11 * wx
                    ev = ex0 * omwy + ex1 * wy
                    col = jnp.full((LANES,), F_PER * l + f, jnp.int32)
                    plsc.store_scatter(out_buf, [pv, col], ev)
                return c_
            lax.fori_loop(jnp.int32(0), jnp.int32(G), cbody, jnp.int32(0))

        pltpu.sync_copy(out_buf, out_hbm.at[pl.ds(base, C)])
        return carry

    lax.fori_loop(jnp.int32(0), jnp.int32(nchunk), chunk_body, jnp.int32(0))


def kernel(xy, emb):
    n = xy.shape[0]
    assert n % (NW * C) == 0
    x = jnp.asarray(xy[:, 0])
    y = jnp.asarray(xy[:, 1])
    emb_flat = emb.reshape(LEVELS * TABLE * F_PER)

    mesh = plsc.VectorSubcoreMesh(core_axis_name="c", subcore_axis_name="s",
                                  num_cores=NC, num_subcores=NS)
    run = pl.kernel(
        functools.partial(_encoder_body, n_points=n),
        out_type=jax.ShapeDtypeStruct((n, LEVELS * F_PER), jnp.float32),
        mesh=mesh,
        compiler_params=pltpu.CompilerParams(needs_layout_passes=False,
                                             use_tc_tiling_on_sc=False),
        scratch_types=[
            pltpu.VMEM((C,), jnp.float32),
            pltpu.VMEM((C,), jnp.float32),
            pltpu.VMEM((C,), jnp.float32),
            pltpu.VMEM((C,), jnp.float32),
            pltpu.VMEM((ROWS, F_PER * C), jnp.int32),
            pltpu.VMEM((ROWS, F_PER * C), jnp.float32),
            pltpu.VMEM((C, LEVELS * F_PER), jnp.float32),
            pltpu.SemaphoreType.DMA,
        ],
    )
    return run(x, y, emb_flat)


# speedup vs baseline: 1.0397x; 1.0397x over previous
import functools

import jax
import jax.numpy as jnp
from jax import lax
from jax.experimental import pallas as pl
from jax.experimental.pallas import tpu as pltpu
from jax.experimental.pallas import tpu_sc as plsc

LEVELS = 16
F_PER = 2
BASE_RES = 16
SCALE = 1.5
TABLE = 524288
MASK = TABLE - 1
RES = [int(BASE_RES * SCALE**l) for l in range(LEVELS)]

NC = 2
NS = 16
NW = NC * NS
LANES = 16

C = 128
G = C // LANES
ROWS = 4 * LEVELS
D = 8


def _encoder_body(x_hbm, y_hbm, emb_hbm, out_hbm,
                  xv, yv, uxv, uyv, idx3, off3, rowsf, out_buf, sem,
                  *, n_points):
    pw = n_points // NW
    nchunk = pw // C
    wid = lax.axis_index("s") * jnp.int32(NC) + lax.axis_index("c")
    base0 = wid * jnp.int32(pw)
    iota = lax.iota(jnp.int32, LANES)

    def chunk_body(ci, carry):
        base = base0 + ci * jnp.int32(C)
        pltpu.sync_copy(x_hbm.at[pl.ds(base, C)], xv)
        pltpu.sync_copy(y_hbm.at[pl.ds(base, C)], yv)

        def ubody(g, c_):
            sl = pl.ds(g * jnp.int32(LANES), LANES)
            uxv[sl] = jnp.clip((xv[sl] + 1.0) * 0.5, 0.0, 1.0)
            uyv[sl] = jnp.clip((yv[sl] + 1.0) * 0.5, 0.0, 1.0)
            return c_
        lax.fori_loop(jnp.int32(0), jnp.int32(G), ubody, jnp.int32(0))

        handles = []
        for l in range(LEVELS):
            resm1 = RES[l] - 1

            def hbody(g, c_, l=l, resm1=resm1):
                sl = pl.ds(g * jnp.int32(LANES), LANES)
                px = uxv[sl] * float(resm1)
                py = uyv[sl] * float(resm1)
                x0 = px.astype(jnp.int32)
                y0 = py.astype(jnp.int32)
                x1 = jnp.minimum(x0 + jnp.int32(1), jnp.int32(resm1))
                y1 = jnp.minimum(y0 + jnp.int32(1), jnp.int32(resm1))
                hx0 = x0 * jnp.int32(73856093)
                hx1 = x1 * jnp.int32(73856093)
                hy0 = y0 * jnp.int32(19349663)
                hy1 = y1 * jnp.int32(19349663)
                off = jnp.int32(l * TABLE)
                msk = jnp.int32(MASK)
                m3 = jnp.int32(3)
                one = jnp.int32(1)
                h00 = ((hx0 ^ hy0) & msk) + off
                h10 = ((hx1 ^ hy0) & msk) + off
                h01 = ((hx0 ^ hy1) & msk) + off
                h11 = ((hx1 ^ hy1) & msk) + off
                idx3[4 * l + 0, sl] = h00 >> jnp.int32(2)
                off3[4 * l + 0, sl] = (h00 & m3) << one
                idx3[4 * l + 1, sl] = h10 >> jnp.int32(2)
                off3[4 * l + 1, sl] = (h10 & m3) << one
                idx3[4 * l + 2, sl] = h01 >> jnp.int32(2)
                off3[4 * l + 2, sl] = (h01 & m3) << one
                idx3[4 * l + 3, sl] = h11 >> jnp.int32(2)
                off3[4 * l + 3, sl] = (h11 & m3) << one
                return c_
            lax.fori_loop(jnp.int32(0), jnp.int32(G), hbody, jnp.int32(0))

            for c in range(4):
                r = 4 * l + c
                h = pltpu.make_async_copy(emb_hbm.at[idx3.at[jnp.int32(r)]],
                                          rowsf.at[jnp.int32(r)], sem)
                h.start()
                handles.append(h)

        for h in handles:
            h.wait()

        for l in range(LEVELS):
            resm1 = RES[l] - 1

            def cbody(g, c_, l=l, resm1=resm1):
                sl = pl.ds(g * jnp.int32(LANES), LANES)
                px = uxv[sl] * float(resm1)
                py = uyv[sl] * float(resm1)
                wx = px - px.astype(jnp.int32).astype(jnp.float32)
                wy = py - py.astype(jnp.int32).astype(jnp.float32)
                omwx = 1.0 - wx
                omwy = 1.0 - wy
                pv = iota + (g * jnp.int32(LANES))
                rr0 = jnp.full((LANES,), 4 * l + 0, jnp.int32)
                rr1 = jnp.full((LANES,), 4 * l + 1, jnp.int32)
                rr2 = jnp.full((LANES,), 4 * l + 2, jnp.int32)
                rr3 = jnp.full((LANES,), 4 * l + 3, jnp.int32)
                o0 = off3[4 * l + 0, sl]
                o1 = off3[4 * l + 1, sl]
                o2 = off3[4 * l + 2, sl]
                o3 = off3[4 * l + 3, sl]
                for f in range(F_PER):
                    fo = jnp.int32(f)
                    e00 = plsc.load_gather(rowsf, [rr0, pv, o0 + fo])
                    e10 = plsc.load_gather(rowsf, [rr1, pv, o1 + fo])
                    e01 = plsc.load_gather(rowsf, [rr2, pv, o2 + fo])
                    e11 = plsc.load_gather(rowsf, [rr3, pv, o3 + fo])
                    ex0 = e00 * omwx + e10 * wx
                    ex1 = e01 * omwx + e11 * wx
                    ev = ex0 * omwy + ex1 * wy
                    col = jnp.full((LANES,), F_PER * l + f, jnp.int32)
                    plsc.store_scatter(out_buf, [pv, col], ev)
                return c_
            lax.fori_loop(jnp.int32(0), jnp.int32(G), cbody, jnp.int32(0))

        pltpu.sync_copy(out_buf, out_hbm.at[pl.ds(base, C)])
        return carry

    lax.fori_loop(jnp.int32(0), jnp.int32(nchunk), chunk_body, jnp.int32(0))


def kernel(xy, emb):
    n = xy.shape[0]
    assert n % (NW * C) == 0
    x = jnp.asarray(xy[:, 0])
    y = jnp.asarray(xy[:, 1])
    emb_rows = emb.reshape(LEVELS * TABLE * F_PER // D, D)

    mesh = plsc.VectorSubcoreMesh(core_axis_name="c", subcore_axis_name="s",
                                  num_cores=NC, num_subcores=NS)
    run = pl.kernel(
        functools.partial(_encoder_body, n_points=n),
        out_type=jax.ShapeDtypeStruct((n, LEVELS * F_PER), jnp.float32),
        mesh=mesh,
        compiler_params=pltpu.CompilerParams(needs_layout_passes=False,
                                             use_tc_tiling_on_sc=False),
        scratch_types=[
            pltpu.VMEM((C,), jnp.float32),
            pltpu.VMEM((C,), jnp.float32),
            pltpu.VMEM((C,), jnp.float32),
            pltpu.VMEM((C,), jnp.float32),
            pltpu.VMEM((ROWS, C), jnp.int32),
            pltpu.VMEM((ROWS, C), jnp.int32),
            pltpu.VMEM((ROWS, C, D), jnp.float32),
            pltpu.VMEM((C, LEVELS * F_PER), jnp.float32),
            pltpu.SemaphoreType.DMA,
        ],
    )
    return run(x, y, emb_rows)


# speedup vs baseline: 8.6141x; 8.2852x over previous
import functools

import jax
import jax.numpy as jnp
from jax import lax
from jax.experimental import pallas as pl
from jax.experimental.pallas import tpu as pltpu
from jax.experimental.pallas import tpu_sc as plsc

LEVELS = 16
F_PER = 2
BASE_RES = 16
SCALE = 1.5
TABLE = 524288
MASK = TABLE - 1
RES = [int(BASE_RES * SCALE**l) for l in range(LEVELS)]

NC = 2
NS = 16
NW = NC * NS
LANES = 16

C = 128
G = C // LANES
ROWS = 4 * LEVELS
D = 8


def _encoder_body(x_hbm, y_hbm, emb_hbm, out_hbm,
                  xv, yv, uxv, uyv, idx3, off3, rowsf, out_buf, sem,
                  *, n_points):
    pw = n_points // NW
    nchunk = pw // C
    wid = lax.axis_index("s") * jnp.int32(NC) + lax.axis_index("c")
    base0 = wid * jnp.int32(pw)
    iota = lax.iota(jnp.int32, LANES)

    def chunk_body(ci, carry):
        base = base0 + ci * jnp.int32(C)
        pltpu.sync_copy(x_hbm.at[pl.ds(base, C)], xv)
        pltpu.sync_copy(y_hbm.at[pl.ds(base, C)], yv)

        def ubody(g, c_):
            sl = pl.ds(g * jnp.int32(LANES), LANES)
            uxv[sl] = jnp.clip((xv[sl] + 1.0) * 0.5, 0.0, 1.0)
            uyv[sl] = jnp.clip((yv[sl] + 1.0) * 0.5, 0.0, 1.0)
            return c_
        lax.fori_loop(jnp.int32(0), jnp.int32(G), ubody, jnp.int32(0))

        handles = []
        for l in range(LEVELS):
            resm1 = RES[l] - 1

            def hbody(g, c_, l=l, resm1=resm1):
                sl = pl.ds(g * jnp.int32(LANES), LANES)
                px = uxv[sl] * float(resm1)
                py = uyv[sl] * float(resm1)
                x0 = px.astype(jnp.int32)
                y0 = py.astype(jnp.int32)
                x1 = jnp.minimum(x0 + jnp.int32(1), jnp.int32(resm1))
                y1 = jnp.minimum(y0 + jnp.int32(1), jnp.int32(resm1))
                hx0 = x0 * jnp.int32(73856093)
                hx1 = x1 * jnp.int32(73856093)
                hy0 = y0 * jnp.int32(19349663)
                hy1 = y1 * jnp.int32(19349663)
                off = jnp.int32(l * TABLE)
                msk = jnp.int32(MASK)
                m3 = jnp.int32(3)
                one = jnp.int32(1)
                h00 = ((hx0 ^ hy0) & msk) + off
                h10 = ((hx1 ^ hy0) & msk) + off
                h01 = ((hx0 ^ hy1) & msk) + off
                h11 = ((hx1 ^ hy1) & msk) + off
                idx3[4 * l + 0, sl] = h00 >> jnp.int32(2)
                off3[4 * l + 0, sl] = (h00 & m3) << one
                idx3[4 * l + 1, sl] = h10 >> jnp.int32(2)
                off3[4 * l + 1, sl] = (h10 & m3) << one
                idx3[4 * l + 2, sl] = h01 >> jnp.int32(2)
                off3[4 * l + 2, sl] = (h01 & m3) << one
                idx3[4 * l + 3, sl] = h11 >> jnp.int32(2)
                off3[4 * l + 3, sl] = (h11 & m3) << one
                return c_
            lax.fori_loop(jnp.int32(0), jnp.int32(G), hbody, jnp.int32(0))

            for c in range(4):
                r = 4 * l + c
                h = pltpu.make_async_copy(emb_hbm.at[idx3.at[jnp.int32(r)]],
                                          rowsf.at[jnp.int32(r)], sem)
                h.start()
                handles.append(h)

        for h in handles:
            h.wait()

        for l in range(LEVELS):
            resm1 = RES[l] - 1

            def cbody(g, c_, l=l, resm1=resm1):
                sl = pl.ds(g * jnp.int32(LANES), LANES)
                px = uxv[sl] * float(resm1)
                py = uyv[sl] * float(resm1)
                wx = px - px.astype(jnp.int32).astype(jnp.float32)
                wy = py - py.astype(jnp.int32).astype(jnp.float32)
                omwx = 1.0 - wx
                omwy = 1.0 - wy
                pv = iota + (g * jnp.int32(LANES))
                rr0 = jnp.full((LANES,), 4 * l + 0, jnp.int32)
                rr1 = jnp.full((LANES,), 4 * l + 1, jnp.int32)
                rr2 = jnp.full((LANES,), 4 * l + 2, jnp.int32)
                rr3 = jnp.full((LANES,), 4 * l + 3, jnp.int32)
                o0 = off3[4 * l + 0, sl]
                o1 = off3[4 * l + 1, sl]
                o2 = off3[4 * l + 2, sl]
                o3 = off3[4 * l + 3, sl]
                for f in range(F_PER):
                    fo = jnp.int32(f)
                    e00 = plsc.load_gather(rowsf, [rr0, pv, o0 + fo])
                    e10 = plsc.load_gather(rowsf, [rr1, pv, o1 + fo])
                    e01 = plsc.load_gather(rowsf, [rr2, pv, o2 + fo])
                    e11 = plsc.load_gather(rowsf, [rr3, pv, o3 + fo])
                    ex0 = e00 * omwx + e10 * wx
                    ex1 = e01 * omwx + e11 * wx
                    ev = ex0 * omwy + ex1 * wy
                    col = jnp.full((LANES,), F_PER * l + f, jnp.int32)
                    plsc.store_scatter(out_buf, [pv, col], ev)
                return c_
            lax.fori_loop(jnp.int32(0), jnp.int32(G), cbody, jnp.int32(0))

        pltpu.sync_copy(out_buf, out_hbm.at[pl.ds(base, C)])
        return carry

    lax.fori_loop(jnp.int32(0), jnp.int32(nchunk), chunk_body, jnp.int32(0))


NBLK = LEVELS * TABLE // 128
BLK_PER_W = NBLK // NW
BLK_CHUNK = 64


def _relayout_body(embp_hbm, out_hbm, inv, outv, *, dummy=None):
    wid = lax.axis_index("s") * jnp.int32(NC) + lax.axis_index("c")
    iota = lax.iota(jnp.int32, LANES)
    nchunk = BLK_PER_W // BLK_CHUNK
    wbase = wid * jnp.int32(BLK_PER_W * 256)

    def chunk_body(ci, carry):
        base = wbase + ci * jnp.int32(BLK_CHUNK * 256)
        pltpu.sync_copy(embp_hbm.at[pl.ds(base, BLK_CHUNK * 256)], inv)

        def bbody(b, c_):
            for k in range(8):
                src0 = pl.ds(b * jnp.int32(256) + jnp.int32(k * LANES), LANES)
                src1 = pl.ds(b * jnp.int32(256) + jnp.int32(128 + k * LANES), LANES)
                f0 = inv[src0]
                f1 = inv[src1]
                pos = b * jnp.int32(256) + ((iota + jnp.int32(k * LANES))
                                            << jnp.int32(1))
                r0 = pos >> jnp.int32(3)
                c0 = pos & jnp.int32(7)
                plsc.store_scatter(outv, [r0, c0], f0)
                pos1 = pos + jnp.int32(1)
                r1 = pos1 >> jnp.int32(3)
                c1 = pos1 & jnp.int32(7)
                plsc.store_scatter(outv, [r1, c1], f1)
            return c_
        lax.fori_loop(jnp.int32(0), jnp.int32(BLK_CHUNK), bbody, jnp.int32(0))

        rbase = (wbase >> jnp.int32(3)) + ci * jnp.int32(BLK_CHUNK * 32)
        pltpu.sync_copy(outv, out_hbm.at[pl.ds(rbase, BLK_CHUNK * 32)])
        return carry

    lax.fori_loop(jnp.int32(0), jnp.int32(nchunk), chunk_body, jnp.int32(0))


def kernel(xy, emb):
    n = xy.shape[0]
    assert n % (NW * C) == 0
    x = jnp.asarray(xy[:, 0])
    y = jnp.asarray(xy[:, 1])
    embp = emb.reshape(LEVELS, TABLE // 128, 128, F_PER)
    embp = embp.transpose(0, 1, 3, 2).reshape(LEVELS * TABLE * F_PER)

    mesh0 = plsc.VectorSubcoreMesh(core_axis_name="c", subcore_axis_name="s",
                                   num_cores=NC, num_subcores=NS)
    relayout = pl.kernel(
        _relayout_body,
        out_type=jax.ShapeDtypeStruct((LEVELS * TABLE * F_PER // D, D),
                                      jnp.float32),
        mesh=mesh0,
        compiler_params=pltpu.CompilerParams(needs_layout_passes=False,
                                             use_tc_tiling_on_sc=False),
        scratch_types=[
            pltpu.VMEM((BLK_CHUNK * 256,), jnp.float32),
            pltpu.VMEM((BLK_CHUNK * 32, D), jnp.float32),
        ],
    )
    emb_rows = relayout(embp)

    mesh = plsc.VectorSubcoreMesh(core_axis_name="c", subcore_axis_name="s",
                                  num_cores=NC, num_subcores=NS)
    run = pl.kernel(
        functools.partial(_encoder_body, n_points=n),
        out_type=jax.ShapeDtypeStruct((n, LEVELS * F_PER), jnp.float32),
        mesh=mesh,
        compiler_params=pltpu.CompilerParams(needs_layout_passes=False,
                                             use_tc_tiling_on_sc=False),
        scratch_types=[
            pltpu.VMEM((C,), jnp.float32),
            pltpu.VMEM((C,), jnp.float32),
            pltpu.VMEM((C,), jnp.float32),
            pltpu.VMEM((C,), jnp.float32),
            pltpu.VMEM((ROWS, C), jnp.int32),
            pltpu.VMEM((ROWS, C), jnp.int32),
            pltpu.VMEM((ROWS, C, D), jnp.float32),
            pltpu.VMEM((C, LEVELS * F_PER), jnp.float32),
            pltpu.SemaphoreType.DMA,
        ],
    )
    return run(x, y, emb_rows)


# speedup vs baseline: 12.0213x; 1.3955x over previous
import functools

import jax
import jax.numpy as jnp
from jax import lax
from jax.experimental import pallas as pl
from jax.experimental.pallas import tpu as pltpu
from jax.experimental.pallas import tpu_sc as plsc

LEVELS = 16
F_PER = 2
BASE_RES = 16
SCALE = 1.5
TABLE = 524288
MASK = TABLE - 1
RES = [int(BASE_RES * SCALE**l) for l in range(LEVELS)]

NC = 2
NS = 16
NW = NC * NS
LANES = 16

C = 128
G = C // LANES
ROWS = 4 * LEVELS
D = 8


def _encoder_body(x_hbm, y_hbm, emb_hbm, out_hbm,
                  xv, yv, uxv, uyv, idx3, off3, rowsf, out_buf, sem,
                  *, n_points):
    pw = n_points // NW
    nchunk = pw // C
    wid = lax.axis_index("s") * jnp.int32(NC) + lax.axis_index("c")
    base0 = wid * jnp.int32(pw)
    iota = lax.iota(jnp.int32, LANES)

    def chunk_body(ci, carry):
        base = base0 + ci * jnp.int32(C)
        pltpu.sync_copy(x_hbm.at[pl.ds(base, C)], xv)
        pltpu.sync_copy(y_hbm.at[pl.ds(base, C)], yv)

        def ubody(g, c_):
            sl = pl.ds(g * jnp.int32(LANES), LANES)
            uxv[sl] = jnp.clip((xv[sl] + 1.0) * 0.5, 0.0, 1.0)
            uyv[sl] = jnp.clip((yv[sl] + 1.0) * 0.5, 0.0, 1.0)
            return c_
        lax.fori_loop(jnp.int32(0), jnp.int32(G), ubody, jnp.int32(0))

        handles = [[] for _ in range(LEVELS)]
        for l in range(LEVELS):
            resm1 = RES[l] - 1

            def hbody(g, c_, l=l, resm1=resm1):
                sl = pl.ds(g * jnp.int32(LANES), LANES)
                px = uxv[sl] * float(resm1)
                py = uyv[sl] * float(resm1)
                x0 = px.astype(jnp.int32)
                y0 = py.astype(jnp.int32)
                x1 = jnp.minimum(x0 + jnp.int32(1), jnp.int32(resm1))
                y1 = jnp.minimum(y0 + jnp.int32(1), jnp.int32(resm1))
                hx0 = x0 * jnp.int32(73856093)
                hx1 = x1 * jnp.int32(73856093)
                hy0 = y0 * jnp.int32(19349663)
                hy1 = y1 * jnp.int32(19349663)
                off = jnp.int32(l * TABLE)
                msk = jnp.int32(MASK)
                m3 = jnp.int32(3)
                one = jnp.int32(1)
                h00 = ((hx0 ^ hy0) & msk) + off
                h10 = ((hx1 ^ hy0) & msk) + off
                h01 = ((hx0 ^ hy1) & msk) + off
                h11 = ((hx1 ^ hy1) & msk) + off
                idx3[4 * l + 0, sl] = h00 >> jnp.int32(2)
                off3[4 * l + 0, sl] = (h00 & m3) << one
                idx3[4 * l + 1, sl] = h10 >> jnp.int32(2)
                off3[4 * l + 1, sl] = (h10 & m3) << one
                idx3[4 * l + 2, sl] = h01 >> jnp.int32(2)
                off3[4 * l + 2, sl] = (h01 & m3) << one
                idx3[4 * l + 3, sl] = h11 >> jnp.int32(2)
                off3[4 * l + 3, sl] = (h11 & m3) << one
                return c_
            lax.fori_loop(jnp.int32(0), jnp.int32(G), hbody, jnp.int32(0))

            for c in range(4):
                r = 4 * l + c
                h = pltpu.make_async_copy(emb_hbm.at[idx3.at[jnp.int32(r)]],
                                          rowsf.at[jnp.int32(r)], sem.at[jnp.int32(l)])
                h.start()
                handles[l].append(h)

        for l in range(LEVELS):
            resm1 = RES[l] - 1
            for h in handles[l]:
                h.wait()

            def cbody(g, c_, l=l, resm1=resm1):
                sl = pl.ds(g * jnp.int32(LANES), LANES)
                px = uxv[sl] * float(resm1)
                py = uyv[sl] * float(resm1)
                wx = px - px.astype(jnp.int32).astype(jnp.float32)
                wy = py - py.astype(jnp.int32).astype(jnp.float32)
                omwx = 1.0 - wx
                omwy = 1.0 - wy
                pv = iota + (g * jnp.int32(LANES))
                rr0 = jnp.full((LANES,), 4 * l + 0, jnp.int32)
                rr1 = jnp.full((LANES,), 4 * l + 1, jnp.int32)
                rr2 = jnp.full((LANES,), 4 * l + 2, jnp.int32)
                rr3 = jnp.full((LANES,), 4 * l + 3, jnp.int32)
                o0 = off3[4 * l + 0, sl]
                o1 = off3[4 * l + 1, sl]
                o2 = off3[4 * l + 2, sl]
                o3 = off3[4 * l + 3, sl]
                for f in range(F_PER):
                    fo = jnp.int32(f)
                    e00 = plsc.load_gather(rowsf, [rr0, pv, o0 + fo])
                    e10 = plsc.load_gather(rowsf, [rr1, pv, o1 + fo])
                    e01 = plsc.load_gather(rowsf, [rr2, pv, o2 + fo])
                    e11 = plsc.load_gather(rowsf, [rr3, pv, o3 + fo])
                    ex0 = e00 * omwx + e10 * wx
                    ex1 = e01 * omwx + e11 * wx
                    ev = ex0 * omwy + ex1 * wy
                    col = jnp.full((LANES,), F_PER * l + f, jnp.int32)
                    plsc.store_scatter(out_buf, [pv, col], ev)
                return c_
            lax.fori_loop(jnp.int32(0), jnp.int32(G), cbody, jnp.int32(0))

        pltpu.sync_copy(out_buf, out_hbm.at[pl.ds(base, C)])
        return carry

    lax.fori_loop(jnp.int32(0), jnp.int32(nchunk), chunk_body, jnp.int32(0))


NBLK = LEVELS * TABLE // 128
BLK_PER_W = NBLK // NW
BLK_CHUNK = 64


def _relayout_body(embp_hbm, out_hbm, inv, outv, *, dummy=None):
    wid = lax.axis_index("s") * jnp.int32(NC) + lax.axis_index("c")
    iota = lax.iota(jnp.int32, LANES)
    nchunk = BLK_PER_W // BLK_CHUNK
    wbase = wid * jnp.int32(BLK_PER_W * 256)

    def chunk_body(ci, carry):
        base = wbase + ci * jnp.int32(BLK_CHUNK * 256)
        pltpu.sync_copy(embp_hbm.at[pl.ds(base, BLK_CHUNK * 256)], inv)

        def bbody(b, c_):
            for k in range(8):
                src0 = pl.ds(b * jnp.int32(256) + jnp.int32(k * LANES), LANES)
                src1 = pl.ds(b * jnp.int32(256) + jnp.int32(128 + k * LANES), LANES)
                f0 = inv[src0]
                f1 = inv[src1]
                pos = b * jnp.int32(256) + ((iota + jnp.int32(k * LANES))
                                            << jnp.int32(1))
                r0 = pos >> jnp.int32(3)
                c0 = pos & jnp.int32(7)
                plsc.store_scatter(outv, [r0, c0], f0)
                pos1 = pos + jnp.int32(1)
                r1 = pos1 >> jnp.int32(3)
                c1 = pos1 & jnp.int32(7)
                plsc.store_scatter(outv, [r1, c1], f1)
            return c_
        lax.fori_loop(jnp.int32(0), jnp.int32(BLK_CHUNK), bbody, jnp.int32(0))

        rbase = (wbase >> jnp.int32(3)) + ci * jnp.int32(BLK_CHUNK * 32)
        pltpu.sync_copy(outv, out_hbm.at[pl.ds(rbase, BLK_CHUNK * 32)])
        return carry

    lax.fori_loop(jnp.int32(0), jnp.int32(nchunk), chunk_body, jnp.int32(0))


def kernel(xy, emb):
    n = xy.shape[0]
    assert n % (NW * C) == 0
    x = jnp.asarray(xy[:, 0])
    y = jnp.asarray(xy[:, 1])
    embp = emb.reshape(LEVELS, TABLE // 128, 128, F_PER)
    embp = embp.transpose(0, 1, 3, 2).reshape(LEVELS * TABLE * F_PER)

    mesh0 = plsc.VectorSubcoreMesh(core_axis_name="c", subcore_axis_name="s",
                                   num_cores=NC, num_subcores=NS)
    relayout = pl.kernel(
        _relayout_body,
        out_type=jax.ShapeDtypeStruct((LEVELS * TABLE * F_PER // D, D),
                                      jnp.float32),
        mesh=mesh0,
        compiler_params=pltpu.CompilerParams(needs_layout_passes=False,
                                             use_tc_tiling_on_sc=False),
        scratch_types=[
            pltpu.VMEM((BLK_CHUNK * 256,), jnp.float32),
            pltpu.VMEM((BLK_CHUNK * 32, D), jnp.float32),
        ],
    )
    emb_rows = relayout(embp)

    mesh = plsc.VectorSubcoreMesh(core_axis_name="c", subcore_axis_name="s",
                                  num_cores=NC, num_subcores=NS)
    run = pl.kernel(
        functools.partial(_encoder_body, n_points=n),
        out_type=jax.ShapeDtypeStruct((n, LEVELS * F_PER), jnp.float32),
        mesh=mesh,
        compiler_params=pltpu.CompilerParams(needs_layout_passes=False,
                                             use_tc_tiling_on_sc=False),
        scratch_types=[
            pltpu.VMEM((C,), jnp.float32),
            pltpu.VMEM((C,), jnp.float32),
            pltpu.VMEM((C,), jnp.float32),
            pltpu.VMEM((C,), jnp.float32),
            pltpu.VMEM((ROWS, C), jnp.int32),
            pltpu.VMEM((ROWS, C), jnp.int32),
            pltpu.VMEM((ROWS, C, D), jnp.float32),
            pltpu.VMEM((C, LEVELS * F_PER), jnp.float32),
            pltpu.SemaphoreType.DMA((LEVELS,)),
        ],
    )
    return run(x, y, emb_rows)
